# baseline (device time: 17405 ns/iter reference)
import jax
import jax.numpy as jnp
from jax import lax
from jax.experimental import pallas as pl
from jax.experimental.pallas import tpu as pltpu

N_DEV = 4
TRIM = 32


def kernel(x, Wq, K_ext, V_ext, Wo):
    B, Sq, Dm = x.shape
    _, Skv_sh, Hq, Dh = K_ext.shape
    HD = Hq * Dh
    R = B * Sq

    x2 = x.reshape(R, Dm)
    K2 = K_ext.reshape(B, Skv_sh, HD)
    V2 = V_ext.reshape(B, Skv_sh, HD)

    def body(x_ref, wq_ref, k_ref, v_ref, wo_ref, out_ref,
             num_parts, ml_parts, send_sems, recv_sems):
        my = lax.axis_index("i")

        def compute_partial(b, nrows):
            r0 = b * Sq
            q = jnp.dot(x_ref[r0:r0 + nrows, :], wq_ref[...],
                        preferred_element_type=jnp.float32)
            qi = lax.broadcasted_iota(jnp.int32, (nrows, Skv_sh), 0)
            ki = my * Skv_sh + lax.broadcasted_iota(
                jnp.int32, (nrows, Skv_sh), 1)
            mask = (jnp.abs(qi - ki) <= 128) | (ki < 32) | (qi < 32)
            kb = k_ref[b]
            vb = v_ref[b]
            for h in range(Hq):
                q_bh = q[:, h * Dh:(h + 1) * Dh]
                k_bh = kb[:, h * Dh:(h + 1) * Dh]
                v_bh = vb[:, h * Dh:(h + 1) * Dh]
                s = lax.dot_general(
                    q_bh, k_bh, (((1,), (1,)), ((), ())),
                    preferred_element_type=jnp.float32,
                ) * 0.125
                s = jnp.where(mask, s, -1e9)
                m = jnp.max(s, axis=-1, keepdims=True)
                p = jnp.exp(s - m)
                l = jnp.sum(p, axis=-1, keepdims=True)
                num = jnp.dot(p, v_bh, preferred_element_type=jnp.float32)
                num_parts[0, r0:r0 + nrows, h * Dh:(h + 1) * Dh] = num
                ml_parts[0, r0:r0 + nrows, h:h + 1] = m
                ml_parts[0, r0:r0 + nrows, Hq + h:Hq + h + 1] = l

        def make_rdmas(b, nrows):
            rdmas = []
            for off in range(1, N_DEV):
                dst = (my + off) % N_DEV
                slot = N_DEV - off
                for buf, parts in enumerate((num_parts, ml_parts)):
                    idx = ((off - 1) * B + b) * 2 + buf
                    rdmas.append(pltpu.make_async_remote_copy(
                        src_ref=parts.at[0, pl.ds(b * Sq, nrows)],
                        dst_ref=parts.at[slot, pl.ds(b * Sq, nrows)],
                        send_sem=send_sems.at[idx],
                        recv_sem=recv_sems.at[idx],
                        device_id=(dst,),
                        device_id_type=pl.DeviceIdType.MESH,
                    ))
            return rdmas

        def recv_rdmas(k, nrows):
            off = N_DEV - k
            rdmas = []
            for b in range(B):
                for buf, parts in enumerate((num_parts, ml_parts)):
                    idx = ((off - 1) * B + b) * 2 + buf
                    rdmas.append(pltpu.make_async_remote_copy(
                        src_ref=parts.at[0, pl.ds(b * Sq, nrows)],
                        dst_ref=parts.at[k, pl.ds(b * Sq, nrows)],
                        send_sem=send_sems.at[idx],
                        recv_sem=recv_sems.at[idx],
                        device_id=(my,),
                        device_id_type=pl.DeviceIdType.MESH,
                    ))
            return rdmas

        full = my < 2

        @pl.when(full)
        def _():
            compute_partial(0, Sq)

        @pl.when(~full)
        def _():
            compute_partial(0, TRIM)

        barrier = pltpu.get_barrier_semaphore()
        for off in range(1, N_DEV):
            peer = (my + off) % N_DEV
            pl.semaphore_signal(
                barrier, inc=1,
                device_id=(peer,), device_id_type=pl.DeviceIdType.MESH,
            )
        pl.semaphore_wait(barrier, N_DEV - 1)

        @pl.when(full)
        def _():
            for r in make_rdmas(0, Sq):
                r.start()
            compute_partial(1, Sq)
            for r in make_rdmas(1, Sq):
                r.start()

        @pl.when(~full)
        def _():
            for r in make_rdmas(0, TRIM):
                r.start()
            compute_partial(1, TRIM)
            for r in make_rdmas(1, TRIM):
                r.start()

        for k in range(1, N_DEV):
            origin = (my + k) % N_DEV

            @pl.when(origin < 2)
            def _(k=k):
                for r in recv_rdmas(k, Sq):
                    r.wait_recv()

            @pl.when(origin >= 2)
            def _(k=k):
                for r in recv_rdmas(k, TRIM):
                    r.wait_recv()

        ri = lax.broadcasted_iota(jnp.int32, (R, 1), 0)
        valid_trim = (ri % Sq) < TRIM

        oks, m_all, l_all = [], [], []
        for k in range(N_DEV):
            origin = (my + k) % N_DEV
            ok = jnp.logical_or(origin < 2, valid_trim)
            oks.append(ok)
            m_all.append(jnp.where(ok, ml_parts[k, :, 0:Hq], -1e9))
            l_all.append(jnp.where(ok, ml_parts[k, :, Hq:2 * Hq], 0.0))

        m_g = m_all[0]
        for k in range(1, N_DEV):
            m_g = jnp.maximum(m_g, m_all[k])
        scales = [jnp.exp(m_all[k] - m_g) for k in range(N_DEV)]
        den = l_all[0] * scales[0]
        for k in range(1, N_DEV):
            den = den + l_all[k] * scales[k]

        ctx_cols = []
        for h in range(Hq):
            num_h = None
            for k in range(N_DEV):
                nk = jnp.where(oks[k], num_parts[k, :, h * Dh:(h + 1) * Dh],
                               0.0) * scales[k][:, h:h + 1]
                num_h = nk if num_h is None else num_h + nk
            ctx_cols.append(num_h / den[:, h:h + 1])
        ctx = jnp.concatenate(ctx_cols, axis=1)

        out_ref[...] = jnp.dot(ctx, wo_ref[...],
                               preferred_element_type=jnp.float32)

        @pl.when(full)
        def _():
            for b in range(B):
                for r in make_rdmas(b, Sq):
                    r.wait_send()

        @pl.when(~full)
        def _():
            for b in range(B):
                for r in make_rdmas(b, TRIM):
                    r.wait_send()

    out = pl.pallas_call(
        body,
        out_shape=jax.ShapeDtypeStruct((R, Dm), jnp.float32),
        in_specs=[pl.BlockSpec(memory_space=pltpu.VMEM)] * 5,
        out_specs=pl.BlockSpec(memory_space=pltpu.VMEM),
        scratch_shapes=[
            pltpu.VMEM((N_DEV, R, HD), jnp.float32),
            pltpu.VMEM((N_DEV, R, 2 * Hq), jnp.float32),
            pltpu.SemaphoreType.DMA((2 * B * (N_DEV - 1),)),
            pltpu.SemaphoreType.DMA((2 * B * (N_DEV - 1),)),
        ],
        compiler_params=pltpu.CompilerParams(collective_id=0),
    )(x2, Wq, K2, V2, Wo)
    return out.reshape(B, Sq, Dm)


# device time: 14702 ns/iter; 1.1839x vs baseline; 1.1839x over previous
import jax
import jax.numpy as jnp
from jax import lax
from jax.experimental import pallas as pl
from jax.experimental.pallas import tpu as pltpu

N_DEV = 4
TRIM = 32


def kernel(x, Wq, K_ext, V_ext, Wo):
    B, Sq, Dm = x.shape
    _, Skv_sh, Hq, Dh = K_ext.shape
    HD = Hq * Dh
    R = B * Sq

    x2 = x.reshape(R, Dm)
    K2 = K_ext.reshape(B, Skv_sh, HD)
    V2 = V_ext.reshape(B, Skv_sh, HD)

    def body(x_ref, wq_ref, k_ref, v_ref, wo_ref, out_ref,
             num_parts, ml_parts, send_sems, recv_sems):
        my = lax.axis_index("i")

        bf16 = jnp.bfloat16

        def compute_partial(b, nrows):
            r0 = b * Sq
            q = jnp.dot(x_ref[r0:r0 + nrows, :].astype(bf16),
                        wq_ref[...].astype(bf16),
                        preferred_element_type=jnp.float32)
            qb = q.astype(bf16)
            qi = lax.broadcasted_iota(jnp.int32, (nrows, Skv_sh), 0)
            ki = my * Skv_sh + lax.broadcasted_iota(
                jnp.int32, (nrows, Skv_sh), 1)
            mask = (jnp.abs(qi - ki) <= 128) | (ki < 32) | (qi < 32)
            kb = k_ref[b].astype(bf16)
            vb = v_ref[b].astype(bf16)
            for h in range(Hq):
                q_bh = qb[:, h * Dh:(h + 1) * Dh]
                k_bh = kb[:, h * Dh:(h + 1) * Dh]
                v_bh = vb[:, h * Dh:(h + 1) * Dh]
                s = lax.dot_general(
                    q_bh, k_bh, (((1,), (1,)), ((), ())),
                    preferred_element_type=jnp.float32,
                ) * 0.125
                s = jnp.where(mask, s, -1e9)
                m = jnp.max(s, axis=-1, keepdims=True)
                p = jnp.exp(s - m)
                l = jnp.sum(p, axis=-1, keepdims=True)
                num = jnp.dot(p.astype(bf16), v_bh,
                              preferred_element_type=jnp.float32)
                num_parts[0, r0:r0 + nrows, h * Dh:(h + 1) * Dh] = (
                    num.astype(bf16))
                ml_parts[0, r0:r0 + nrows, h:h + 1] = m
                ml_parts[0, r0:r0 + nrows, Hq + h:Hq + h + 1] = l

        def make_rdmas(b, nrows):
            rdmas = []
            for off in range(1, N_DEV):
                dst = (my + off) % N_DEV
                slot = N_DEV - off
                for buf, parts in enumerate((num_parts, ml_parts)):
                    idx = ((off - 1) * B + b) * 2 + buf
                    rdmas.append(pltpu.make_async_remote_copy(
                        src_ref=parts.at[0, pl.ds(b * Sq, nrows)],
                        dst_ref=parts.at[slot, pl.ds(b * Sq, nrows)],
                        send_sem=send_sems.at[idx],
                        recv_sem=recv_sems.at[idx],
                        device_id=(dst,),
                        device_id_type=pl.DeviceIdType.MESH,
                    ))
            return rdmas

        def recv_rdmas(k, nrows):
            off = N_DEV - k
            rdmas = []
            for b in range(B):
                for buf, parts in enumerate((num_parts, ml_parts)):
                    idx = ((off - 1) * B + b) * 2 + buf
                    rdmas.append(pltpu.make_async_remote_copy(
                        src_ref=parts.at[0, pl.ds(b * Sq, nrows)],
                        dst_ref=parts.at[k, pl.ds(b * Sq, nrows)],
                        send_sem=send_sems.at[idx],
                        recv_sem=recv_sems.at[idx],
                        device_id=(my,),
                        device_id_type=pl.DeviceIdType.MESH,
                    ))
            return rdmas

        full = my < 2

        barrier = pltpu.get_barrier_semaphore()
        for off in range(1, N_DEV):
            peer = (my + off) % N_DEV
            pl.semaphore_signal(
                barrier, inc=1,
                device_id=(peer,), device_id_type=pl.DeviceIdType.MESH,
            )

        @pl.when(full)
        def _():
            compute_partial(0, Sq)

        @pl.when(~full)
        def _():
            compute_partial(0, TRIM)

        pl.semaphore_wait(barrier, N_DEV - 1)

        @pl.when(full)
        def _():
            for r in make_rdmas(0, Sq):
                r.start()
            compute_partial(1, Sq)
            for r in make_rdmas(1, Sq):
                r.start()

        @pl.when(~full)
        def _():
            for r in make_rdmas(0, TRIM):
                r.start()
            compute_partial(1, TRIM)
            for r in make_rdmas(1, TRIM):
                r.start()

        for k in range(1, N_DEV):
            origin = (my + k) % N_DEV

            @pl.when(origin < 2)
            def _(k=k):
                for r in recv_rdmas(k, Sq):
                    r.wait_recv()

            @pl.when(origin >= 2)
            def _(k=k):
                for r in recv_rdmas(k, TRIM):
                    r.wait_recv()

        ri = lax.broadcasted_iota(jnp.int32, (R, 1), 0)
        valid_trim = (ri % Sq) < TRIM

        oks, m_all, l_all = [], [], []
        for k in range(N_DEV):
            origin = (my + k) % N_DEV
            ok = jnp.logical_or(origin < 2, valid_trim)
            oks.append(ok)
            m_all.append(jnp.where(ok, ml_parts[k, :, 0:Hq], -1e9))
            l_all.append(jnp.where(ok, ml_parts[k, :, Hq:2 * Hq], 0.0))

        m_g = m_all[0]
        for k in range(1, N_DEV):
            m_g = jnp.maximum(m_g, m_all[k])
        scales = [jnp.exp(m_all[k] - m_g) for k in range(N_DEV)]
        den = l_all[0] * scales[0]
        for k in range(1, N_DEV):
            den = den + l_all[k] * scales[k]

        ctx_cols = []
        for h in range(Hq):
            num_h = None
            for k in range(N_DEV):
                nk = jnp.where(
                    oks[k],
                    num_parts[k, :, h * Dh:(h + 1) * Dh].astype(jnp.float32),
                    0.0,
                ) * scales[k][:, h:h + 1]
                num_h = nk if num_h is None else num_h + nk
            ctx_cols.append(num_h / den[:, h:h + 1])
        ctx = jnp.concatenate(ctx_cols, axis=1)

        out_ref[...] = jnp.dot(ctx.astype(bf16), wo_ref[...].astype(bf16),
                               preferred_element_type=jnp.float32)

        @pl.when(full)
        def _():
            for b in range(B):
                for r in make_rdmas(b, Sq):
                    r.wait_send()

        @pl.when(~full)
        def _():
            for b in range(B):
                for r in make_rdmas(b, TRIM):
                    r.wait_send()

    out = pl.pallas_call(
        body,
        out_shape=jax.ShapeDtypeStruct((R, Dm), jnp.float32),
        in_specs=[pl.BlockSpec(memory_space=pltpu.VMEM)] * 5,
        out_specs=pl.BlockSpec(memory_space=pltpu.VMEM),
        scratch_shapes=[
            pltpu.VMEM((N_DEV, R, HD), jnp.bfloat16),
            pltpu.VMEM((N_DEV, R, 2 * Hq), jnp.float32),
            pltpu.SemaphoreType.DMA((2 * B * (N_DEV - 1),)),
            pltpu.SemaphoreType.DMA((2 * B * (N_DEV - 1),)),
        ],
        compiler_params=pltpu.CompilerParams(collective_id=0),
    )(x2, Wq, K2, V2, Wo)
    return out.reshape(B, Sq, Dm)
